# Initial kernel scaffold; baseline (speedup 1.0000x reference)
import jax
import jax.numpy as jnp
from jax.experimental import pallas as pl


def kernel(global_tokenized_prompts, token_embedding):
    raise NotImplementedError("write your pallas kernel here")


# speedup vs baseline: 1.1981x; 1.1981x over previous
import functools

import jax
import jax.numpy as jnp
from jax import lax
from jax.experimental import pallas as pl
from jax.experimental.pallas import tpu as pltpu
from jax.experimental.pallas import tpu_sc as plsc

NUM_CLASSES = 1000
CONTEXT_LENGTH = 77
EMBED_DIM = 512

_B = NUM_CLASSES * CONTEXT_LENGTH
_C = 56
_NC = 2
_NS = 16
_NW = _NC * _NS
_KMAX = 43
_BPW = _KMAX * _C
_IDX_PAD = _NW * _BPW


def _build_gather(vocab, dim):
    mesh = plsc.VectorSubcoreMesh(
        core_axis_name="c", subcore_axis_name="s",
        num_cores=_NC, num_subcores=_NS,
    )

    @functools.partial(
        pl.kernel,
        out_type=jax.ShapeDtypeStruct((_B, dim), jnp.float32),
        mesh=mesh,
        scratch_types=[
            pltpu.VMEM((_BPW,), jnp.int32),
            pltpu.VMEM((2, _C, dim), jnp.float32),
            pltpu.SemaphoreType.DMA,
            pltpu.SemaphoreType.DMA,
            pltpu.SemaphoreType.DMA,
            pltpu.SemaphoreType.DMA,
        ],
    )
    def gather_kernel(table_hbm, idx_hbm, out_hbm, idx_v, rows_v,
                      gsem0, gsem1, osem0, osem1):
        wid = lax.axis_index("s") * _NC + lax.axis_index("c")
        base = wid * _BPW

        pltpu.sync_copy(idx_hbm.at[pl.ds(base, _BPW)], idx_v)

        def start_gather(k, buf, sem):
            return pltpu.async_copy(
                table_hbm.at[idx_v.at[pl.ds(k * _C, _C)]], rows_v.at[buf],
                sem)

        def start_out(k, buf, sem):
            return pltpu.async_copy(
                rows_v.at[buf], out_hbm.at[pl.ds(base + k * _C, _C)], sem)

        def pair(p, _):
            k0 = 2 * p
            k1 = k0 + 1
            g0 = start_gather(k0, 0, gsem0)
            g1 = start_gather(k1, 1, gsem1)
            g0.wait()
            o0 = start_out(k0, 0, osem0)
            g1.wait()
            o1 = start_out(k1, 1, osem1)
            o0.wait()
            o1.wait()
            return 0

        lax.fori_loop(0, 21, pair, 0, unroll=False)

        @pl.when(wid < _NW - 1)
        def _():
            g = start_gather(_KMAX - 1, 0, gsem0)
            g.wait()
            o = start_out(_KMAX - 1, 0, osem0)
            o.wait()

    return gather_kernel


_gather = _build_gather(49408, EMBED_DIM)


def kernel(global_tokenized_prompts, token_embedding):
    idx = global_tokenized_prompts.reshape(-1).astype(jnp.int32)
    idx = jnp.concatenate(
        [idx, jnp.zeros((_IDX_PAD - _B,), jnp.int32)])
    out = _gather(token_embedding, idx)
    return out.reshape(NUM_CLASSES, CONTEXT_LENGTH, EMBED_DIM)


# speedup vs baseline: 1.2073x; 1.0077x over previous
import functools

import jax
import jax.numpy as jnp
from jax import lax
from jax.experimental import pallas as pl
from jax.experimental.pallas import tpu as pltpu
from jax.experimental.pallas import tpu_sc as plsc

NUM_CLASSES = 1000
CONTEXT_LENGTH = 77
EMBED_DIM = 512

_B = NUM_CLASSES * CONTEXT_LENGTH
_C = 56
_NC = 2
_NS = 16
_NW = _NC * _NS
_KMAX = 43
_NBUF = 4
_BPW = _KMAX * _C
_IDX_PAD = _NW * _BPW


def _build_gather(vocab, dim):
    mesh = plsc.VectorSubcoreMesh(
        core_axis_name="c", subcore_axis_name="s",
        num_cores=_NC, num_subcores=_NS,
    )

    @functools.partial(
        pl.kernel,
        out_type=jax.ShapeDtypeStruct((_B, dim), jnp.float32),
        mesh=mesh,
        scratch_types=[
            pltpu.VMEM((_BPW,), jnp.int32),
            pltpu.VMEM((_NBUF, _C, dim), jnp.float32),
            [pltpu.SemaphoreType.DMA] * _NBUF,
            [pltpu.SemaphoreType.DMA] * _NBUF,
        ],
    )
    def gather_kernel(table_hbm, idx_hbm, out_hbm, idx_v, rows_v,
                      gsems, osems):
        wid = lax.axis_index("s") * _NC + lax.axis_index("c")
        base = wid * _BPW
        n_valid = jnp.where(wid < _NW - 1, _KMAX, _KMAX - 1)

        pltpu.sync_copy(idx_hbm.at[pl.ds(base, _BPW)], idx_v)

        def gather_desc(k, buf, sem):
            return pltpu.make_async_copy(
                table_hbm.at[idx_v.at[pl.ds(k * _C, _C)]], rows_v.at[buf],
                sem)

        def out_desc(k, buf, sem):
            return pltpu.make_async_copy(
                rows_v.at[buf], out_hbm.at[pl.ds(base + k * _C, _C)], sem)

        def ring(p, _):
            for b in range(_NBUF):
                k = _NBUF * p + b

                @pl.when(k < n_valid)
                def _():
                    @pl.when(k >= _NBUF)
                    def _():
                        out_desc(k - _NBUF, b, osems[b]).wait()
                    gather_desc(k, b, gsems[b]).start()

            for b in range(_NBUF):
                k = _NBUF * p + b

                @pl.when(k < n_valid)
                def _():
                    gather_desc(k, b, gsems[b]).wait()
                    out_desc(k, b, osems[b]).start()

            return 0

        lax.fori_loop(0, (_KMAX + _NBUF - 1) // _NBUF, ring, 0,
                      unroll=False)

        for b in range(_NBUF):
            out_desc(0, b, osems[b]).wait()

    return gather_kernel


_gather = _build_gather(49408, EMBED_DIM)


def kernel(global_tokenized_prompts, token_embedding):
    idx = global_tokenized_prompts.reshape(-1).astype(jnp.int32)
    idx = jnp.concatenate(
        [idx, jnp.zeros((_IDX_PAD - _B,), jnp.int32)])
    out = _gather(token_embedding, idx)
    return out.reshape(NUM_CLASSES, CONTEXT_LENGTH, EMBED_DIM)


# speedup vs baseline: 7.1941x; 5.9586x over previous
import functools

import jax
import jax.numpy as jnp
from jax import lax
from jax.experimental import pallas as pl
from jax.experimental.pallas import tpu as pltpu
from jax.experimental.pallas import tpu_sc as plsc

NUM_CLASSES = 1000
CONTEXT_LENGTH = 77
EMBED_DIM = 512

_B = NUM_CLASSES * CONTEXT_LENGTH
_C = 56
_NC = 2
_NS = 16
_NW = _NC * _NS
_KMAX = 43
_NBUF = 4
_BPW = _KMAX * _C
_IDX_PAD = _NW * _BPW


def _build_gather(vocab, dim):
    mesh = plsc.VectorSubcoreMesh(
        core_axis_name="c", subcore_axis_name="s",
        num_cores=_NC, num_subcores=_NS,
    )

    @functools.partial(
        pl.kernel,
        out_type=jax.ShapeDtypeStruct((_B, dim), jnp.float32),
        mesh=mesh,
        scratch_types=[
            pltpu.VMEM((_BPW,), jnp.int32),
            pltpu.VMEM((_NBUF, _C, dim), jnp.float32),
            [pltpu.SemaphoreType.DMA] * _NBUF,
            [pltpu.SemaphoreType.DMA] * _NBUF,
        ],
    )
    def gather_kernel(table_hbm, idx_hbm, out_hbm, idx_v, rows_v,
                      gsems, osems):
        wid = lax.axis_index("s") * _NC + lax.axis_index("c")
        base = wid * _BPW
        n_valid = jnp.where(wid < _NW - 1, _KMAX, _KMAX - 1)

        pltpu.sync_copy(idx_hbm.at[pl.ds(base, _BPW)], idx_v)

        def gather_desc(k, buf, sem):
            return pltpu.make_async_copy(
                table_hbm.at[idx_v.at[pl.ds(k * _C, _C)]], rows_v.at[buf],
                sem)

        def out_desc(k, buf, sem):
            return pltpu.make_async_copy(
                rows_v.at[buf], out_hbm.at[pl.ds(base + k * _C, _C)], sem)

        def ring(p, _):
            for b in range(_NBUF):
                k = _NBUF * p + b

                @pl.when(k < n_valid)
                def _():
                    @pl.when(k >= _NBUF)
                    def _():
                        out_desc(k - _NBUF, b, osems[b]).wait()
                    gather_desc(k, b, gsems[b]).start()

            for b in range(_NBUF):
                k = _NBUF * p + b

                @pl.when(k < n_valid)
                def _():
                    gather_desc(k, b, gsems[b]).wait()
                    out_desc(k, b, osems[b]).start()

            return 0

        lax.fori_loop(0, (_KMAX + _NBUF - 1) // _NBUF, ring, 0,
                      unroll=False)

        for b in range(_NBUF):
            out_desc(0, b, osems[b]).wait()

    return gather_kernel


_gather = _build_gather(49408, EMBED_DIM)


def kernel(global_tokenized_prompts, token_embedding):
    idx = global_tokenized_prompts.astype(jnp.int32).T.reshape(-1)
    idx = jnp.concatenate(
        [idx, jnp.zeros((_IDX_PAD - _B,), jnp.int32)])
    out = _gather(token_embedding, idx)
    return out.reshape(CONTEXT_LENGTH, NUM_CLASSES, EMBED_DIM).transpose(
        1, 0, 2)
